# speedup vs baseline: 2.0051x; 2.0051x over previous
import functools

import jax
import jax.numpy as jnp
from jax.experimental import pallas as pl
from jax.experimental.pallas import tpu as pltpu

NUM_CLASSES = 1000
BATCH = 16384
CHUNK = 512
NUM_CHUNKS = BATCH // CHUNK


def _centroid_body(logits_ref, targets_ref, storage_ref, count_ref,
                   storage_out_ref, count_out_ref):
    step = pl.program_id(0)

    x = logits_ref[...]
    m = jnp.max(x, axis=1, keepdims=True)
    e = jnp.exp(x - m)
    s = jnp.sum(e, axis=1, keepdims=True)
    probs = e / s

    t = targets_ref[0, 0, :]
    class_ids = jax.lax.broadcasted_iota(jnp.int32, (NUM_CLASSES, CHUNK), 0)
    one_hot_t = (class_ids == t[None, :]).astype(jnp.float32)

    contrib = jax.lax.dot_general(
        one_hot_t, probs, (((1,), (0,)), ((), ())),
        preferred_element_type=jnp.float32)
    count_part = jnp.sum(one_hot_t, axis=1)[None, :]

    @pl.when(step == 0)
    def _init():
        storage_out_ref[...] = storage_ref[...] + contrib
        count_out_ref[...] = count_ref[...] + count_part

    @pl.when(step != 0)
    def _acc():
        storage_out_ref[...] += contrib
        count_out_ref[...] += count_part


@jax.jit
def kernel(logits, targets, storage, count):
    targets3 = targets.reshape(NUM_CHUNKS, 1, CHUNK)
    count2 = count.reshape(1, NUM_CLASSES)
    storage_out, count_out = pl.pallas_call(
        _centroid_body,
        grid=(NUM_CHUNKS,),
        in_specs=[
            pl.BlockSpec((CHUNK, NUM_CLASSES), lambda i: (i, 0)),
            pl.BlockSpec((1, 1, CHUNK), lambda i: (i, 0, 0)),
            pl.BlockSpec((NUM_CLASSES, NUM_CLASSES), lambda i: (0, 0)),
            pl.BlockSpec((1, NUM_CLASSES), lambda i: (0, 0)),
        ],
        out_specs=[
            pl.BlockSpec((NUM_CLASSES, NUM_CLASSES), lambda i: (0, 0)),
            pl.BlockSpec((1, NUM_CLASSES), lambda i: (0, 0)),
        ],
        out_shape=[
            jax.ShapeDtypeStruct((NUM_CLASSES, NUM_CLASSES), jnp.float32),
            jax.ShapeDtypeStruct((1, NUM_CLASSES), jnp.float32),
        ],
    )(logits, targets3, storage, count2)
    return storage_out, count_out.reshape(NUM_CLASSES)
